# baseline (speedup 1.0000x reference)
import functools

import jax
import jax.numpy as jnp
from jax import lax
from jax.experimental import pallas as pl
from jax.experimental.pallas import tpu as pltpu
from jax.experimental.pallas import tpu_sc as plsc

N_NODES_K = 10000
N_EDGES_K = 320000
D_FEAT_K = 128
NUM_TILES = 32
F_PER_TILE = D_FEAT_K // NUM_TILES
LANES = 16

E_PAD = 327680
CH_A = 20000
CH_C = 10000

_SC_MESH = dict(core_axis_name="c", subcore_axis_name="s")
_SC_PARAMS = pltpu.CompilerParams(needs_layout_passes=False)


def _tile_id():
    return lax.axis_index("s") * 2 + lax.axis_index("c")


def _sign0(wid):
    s = jnp.where(wid == 0, jnp.float32(-1.0), jnp.float32(1.0))
    return lax.broadcast_in_dim(s, (LANES,), ())


def _unpack_uv(pp):
    uu = lax.shift_right_logical(pp, 14)
    vv = pp & jnp.int32(16383)
    return uu, vv


def _inner_products(xt, pack):
    mesh = plsc.VectorSubcoreMesh(**_SC_MESH)

    @functools.partial(
        pl.kernel,
        out_type=jax.ShapeDtypeStruct((NUM_TILES * E_PAD,), jnp.float32),
        mesh=mesh,
        scratch_types=[
            pltpu.VMEM((F_PER_TILE, N_NODES_K), jnp.float32),
            pltpu.VMEM((CH_A,), jnp.int32),
            pltpu.VMEM((CH_A,), jnp.int32),
            pltpu.VMEM((CH_A,), jnp.float32),
            pltpu.SemaphoreType.DMA,
            pltpu.SemaphoreType.DMA,
        ],
        compiler_params=_SC_PARAMS,
    )
    def k(pk_hbm, xt_hbm, out_hbm, xc, pb0, pb1, part, s0, s1):
        wid = _tile_id()
        pltpu.sync_copy(xt_hbm.at[pl.ds(wid * F_PER_TILE, F_PER_TILE)], xc)
        sgn0 = _sign0(wid)
        f_idx = [jnp.full((LANES,), f, jnp.int32) for f in range(F_PER_TILE)]
        n_ch = N_EDGES_K // CH_A

        def cp(ci, pb, sem):
            return pltpu.make_async_copy(pk_hbm.at[pl.ds(ci * CH_A, CH_A)],
                                         pb, sem)

        def do_chunk(ci, pb):
            @plsc.parallel_loop(0, CH_A // LANES, unroll=4)
            def _grp(g):
                uu, vv = _unpack_uv(pb[pl.ds(g * LANES, LANES)])
                cu = plsc.load_gather(xc, [f_idx[0], uu])
                cv = plsc.load_gather(xc, [f_idx[0], vv])
                acc = cu * cv * sgn0
                for f in range(1, F_PER_TILE):
                    cu = plsc.load_gather(xc, [f_idx[f], uu])
                    cv = plsc.load_gather(xc, [f_idx[f], vv])
                    acc = acc + cu * cv
                part[pl.ds(g * LANES, LANES)] = acc

            pltpu.sync_copy(part,
                            out_hbm.at[pl.ds(wid * E_PAD + ci * CH_A, CH_A)])

        cp(0, pb0, s0).start()

        @pl.loop(0, n_ch // 2)
        def _pair(p):
            ci0 = p * 2
            cp(ci0 + 1, pb1, s1).start()
            cp(ci0, pb0, s0).wait()
            do_chunk(ci0, pb0)

            @pl.when(p + 1 < n_ch // 2)
            def _():
                cp(ci0 + 2, pb0, s0).start()

            cp(ci0 + 1, pb1, s1).wait()
            do_chunk(ci0 + 1, pb1)

    return k(pack, xt)


def _factor_energy(innerp_flat):
    blk = 40960
    nblk = E_PAD // blk

    def body(*refs):
        ip_refs = refs[:NUM_TILES]
        f_ref, e_ref = refs[NUM_TILES:]
        i = pl.program_id(0)
        inner = ip_refs[0][...]
        for t in range(1, NUM_TILES):
            inner = inner + ip_refs[t][...]
        a = jnp.maximum(-inner, jnp.float32(1.0 + 1e-7))
        s = jnp.sqrt(a * a - 1.0)
        dist = jnp.log(a + s)
        e = jnp.where(dist > 10.0, dist - 0.69314718, jnp.log(a))
        valid = i * blk + lax.iota(jnp.int32, blk) < N_EDGES_K
        f_ref[...] = jnp.where(valid, -(s / a) / (s + 1e-9), 0.0)

        @pl.when(i == 0)
        def _():
            e_ref[...] = jnp.zeros_like(e_ref)

        e_ref[...] = e_ref[...] + jnp.sum(jnp.where(valid, e, 0.0)).reshape(1)

    def idx_map(i, t=0):
        return (t * nblk + i,)

    return pl.pallas_call(
        body,
        grid=(nblk,),
        in_specs=[pl.BlockSpec((blk,), functools.partial(idx_map, t=t))
                  for t in range(NUM_TILES)],
        out_specs=[
            pl.BlockSpec((blk,), lambda i: (i,)),
            pl.BlockSpec((1,), lambda i: (0,)),
        ],
        out_shape=[
            jax.ShapeDtypeStruct((E_PAD,), jnp.float32),
            jax.ShapeDtypeStruct((1,), jnp.float32),
        ],
    )(*([innerp_flat] * NUM_TILES))


def _scatter_grad(xt, pack, factor):
    mesh = plsc.VectorSubcoreMesh(**_SC_MESH)

    @functools.partial(
        pl.kernel,
        out_type=jax.ShapeDtypeStruct((D_FEAT_K, N_NODES_K), jnp.float32),
        mesh=mesh,
        scratch_types=[
            pltpu.VMEM((F_PER_TILE, N_NODES_K), jnp.float32),
            pltpu.VMEM((F_PER_TILE, N_NODES_K), jnp.float32),
            pltpu.VMEM((CH_C,), jnp.int32),
            pltpu.VMEM((CH_C,), jnp.float32),
            pltpu.VMEM((CH_C,), jnp.int32),
            pltpu.VMEM((CH_C,), jnp.float32),
            pltpu.SemaphoreType.DMA,
            pltpu.SemaphoreType.DMA,
        ],
        compiler_params=_SC_PARAMS,
    )
    def k(pk_hbm, xt_hbm, fac_hbm, out_hbm,
          xc, gc, pb0, fb0, pb1, fb1, s0, s1):
        wid = _tile_id()
        pltpu.sync_copy(xt_hbm.at[pl.ds(wid * F_PER_TILE, F_PER_TILE)], xc)
        sgn0 = _sign0(wid)
        f_idx = [jnp.full((LANES,), f, jnp.int32) for f in range(F_PER_TILE)]
        z = jnp.zeros((LANES,), jnp.float32)
        n_ch = N_EDGES_K // CH_C

        def cps(ci, pb, fb, sem):
            base = ci * CH_C
            return (pltpu.make_async_copy(pk_hbm.at[pl.ds(base, CH_C)], pb, sem),
                    pltpu.make_async_copy(fac_hbm.at[pl.ds(base, CH_C)], fb, sem))

        def do_chunk(pb, fb):
            @plsc.parallel_loop(0, CH_C // LANES, unroll=1)
            def _grp(g):
                uu, vv = _unpack_uv(pb[pl.ds(g * LANES, LANES)])
                ff = fb[pl.ds(g * LANES, LANES)]
                fj = ff * sgn0
                cu = plsc.load_gather(xc, [f_idx[0], uu])
                cv = plsc.load_gather(xc, [f_idx[0], vv])
                plsc.addupdate_scatter(gc, [f_idx[0], vv], fj * cu)
                plsc.addupdate_scatter(gc, [f_idx[0], uu], fj * cv)
                for f in range(1, F_PER_TILE):
                    cu = plsc.load_gather(xc, [f_idx[f], uu])
                    cv = plsc.load_gather(xc, [f_idx[f], vv])
                    plsc.addupdate_scatter(gc, [f_idx[f], vv], ff * cu)
                    plsc.addupdate_scatter(gc, [f_idx[f], uu], ff * cv)

        for c in cps(0, pb0, fb0, s0):
            c.start()

        for f in range(F_PER_TILE):
            @pl.loop(0, N_NODES_K // LANES)
            def _zero(i, f=f):
                gc[f, pl.ds(i * LANES, LANES)] = z

        @pl.loop(0, n_ch // 2)
        def _pair(p):
            ci0 = p * 2
            for c in cps(ci0 + 1, pb1, fb1, s1):
                c.start()
            for c in cps(ci0, pb0, fb0, s0):
                c.wait()
            do_chunk(pb0, fb0)

            @pl.when(p + 1 < n_ch // 2)
            def _():
                for c in cps(ci0 + 2, pb0, fb0, s0):
                    c.start()

            for c in cps(ci0 + 1, pb1, fb1, s1):
                c.wait()
            do_chunk(pb1, fb1)

        pltpu.sync_copy(gc, out_hbm.at[pl.ds(wid * F_PER_TILE, F_PER_TILE)])

    return k(pack, xt, factor)


def kernel(x, edges):
    pack = edges[:, 0] * 16384 + edges[:, 1]
    xt = x.T

    innerp_flat = _inner_products(xt, pack)
    factor, energy = _factor_energy(innerp_flat)
    gradt = _scatter_grad(xt, pack, factor)
    return energy[0], gradt.T


# speedup vs baseline: 1.0051x; 1.0051x over previous
import functools

import jax
import jax.numpy as jnp
from jax import lax
from jax.experimental import pallas as pl
from jax.experimental.pallas import tpu as pltpu
from jax.experimental.pallas import tpu_sc as plsc

N_NODES_K = 10000
N_EDGES_K = 320000
D_FEAT_K = 128
NUM_TILES = 32
F_PER_TILE = D_FEAT_K // NUM_TILES
LANES = 16

E_PAD = 327680
CH_A = 20000
CH_C = 10000

_SC_MESH = dict(core_axis_name="c", subcore_axis_name="s")
_SC_PARAMS = pltpu.CompilerParams(needs_layout_passes=False)


def _tile_id():
    return lax.axis_index("s") * 2 + lax.axis_index("c")


def _sign0(wid):
    s = jnp.where(wid == 0, jnp.float32(-1.0), jnp.float32(1.0))
    return lax.broadcast_in_dim(s, (LANES,), ())


def _unpack_uv(pp):
    uu = lax.shift_right_logical(pp, 14)
    vv = pp & jnp.int32(16383)
    return uu, vv


def _inner_products(xt, pack):
    mesh = plsc.VectorSubcoreMesh(**_SC_MESH)

    @functools.partial(
        pl.kernel,
        out_type=jax.ShapeDtypeStruct((NUM_TILES * E_PAD,), jnp.float32),
        mesh=mesh,
        scratch_types=[
            pltpu.VMEM((F_PER_TILE, N_NODES_K), jnp.float32),
            pltpu.VMEM((CH_A,), jnp.int32),
            pltpu.VMEM((CH_A,), jnp.int32),
            pltpu.VMEM((CH_A,), jnp.float32),
            pltpu.SemaphoreType.DMA,
            pltpu.SemaphoreType.DMA,
        ],
        compiler_params=_SC_PARAMS,
    )
    def k(pk_hbm, xt_hbm, out_hbm, xc, pb0, pb1, part, s0, s1):
        wid = _tile_id()
        pltpu.sync_copy(xt_hbm.at[pl.ds(wid * F_PER_TILE, F_PER_TILE)], xc)
        sgn0 = _sign0(wid)
        f_idx = [jnp.full((LANES,), f, jnp.int32) for f in range(F_PER_TILE)]
        n_ch = N_EDGES_K // CH_A

        def cp(ci, pb, sem):
            return pltpu.make_async_copy(pk_hbm.at[pl.ds(ci * CH_A, CH_A)],
                                         pb, sem)

        def do_chunk(ci, pb):
            @plsc.parallel_loop(0, CH_A // LANES, unroll=4)
            def _grp(g):
                uu, vv = _unpack_uv(pb[pl.ds(g * LANES, LANES)])
                cu = plsc.load_gather(xc, [f_idx[0], uu])
                cv = plsc.load_gather(xc, [f_idx[0], vv])
                acc = cu * cv * sgn0
                for f in range(1, F_PER_TILE):
                    cu = plsc.load_gather(xc, [f_idx[f], uu])
                    cv = plsc.load_gather(xc, [f_idx[f], vv])
                    acc = acc + cu * cv
                part[pl.ds(g * LANES, LANES)] = acc

            pltpu.sync_copy(part,
                            out_hbm.at[pl.ds(wid * E_PAD + ci * CH_A, CH_A)])

        cp(0, pb0, s0).start()

        @pl.loop(0, n_ch // 2)
        def _pair(p):
            ci0 = p * 2
            cp(ci0 + 1, pb1, s1).start()
            cp(ci0, pb0, s0).wait()
            do_chunk(ci0, pb0)

            @pl.when(p + 1 < n_ch // 2)
            def _():
                cp(ci0 + 2, pb0, s0).start()

            cp(ci0 + 1, pb1, s1).wait()
            do_chunk(ci0 + 1, pb1)

    return k(pack, xt)


def _factor_energy(innerp_flat):
    blk = 40960
    nblk = E_PAD // blk

    def body(*refs):
        ip_refs = refs[:NUM_TILES]
        f_ref, e_ref = refs[NUM_TILES:]
        i = pl.program_id(0)
        inner = ip_refs[0][...]
        for t in range(1, NUM_TILES):
            inner = inner + ip_refs[t][...]
        a = jnp.maximum(-inner, jnp.float32(1.0 + 1e-7))
        s = jnp.sqrt(a * a - 1.0)
        dist = jnp.log(a + s)
        e = jnp.where(dist > 10.0, dist - 0.69314718, jnp.log(a))
        valid = i * blk + lax.iota(jnp.int32, blk) < N_EDGES_K
        f_ref[...] = jnp.where(valid, -(s / a) / (s + 1e-9), 0.0)

        @pl.when(i == 0)
        def _():
            e_ref[...] = jnp.zeros_like(e_ref)

        e_ref[...] = e_ref[...] + jnp.sum(jnp.where(valid, e, 0.0)).reshape(1)

    def idx_map(i, t=0):
        return (t * nblk + i,)

    return pl.pallas_call(
        body,
        grid=(nblk,),
        in_specs=[pl.BlockSpec((blk,), functools.partial(idx_map, t=t))
                  for t in range(NUM_TILES)],
        out_specs=[
            pl.BlockSpec((blk,), lambda i: (i,)),
            pl.BlockSpec((1,), lambda i: (0,)),
        ],
        out_shape=[
            jax.ShapeDtypeStruct((E_PAD,), jnp.float32),
            jax.ShapeDtypeStruct((1,), jnp.float32),
        ],
    )(*([innerp_flat] * NUM_TILES))


def _scatter_grad(xt, pack, factor):
    mesh = plsc.VectorSubcoreMesh(**_SC_MESH)

    @functools.partial(
        pl.kernel,
        out_type=jax.ShapeDtypeStruct((D_FEAT_K, N_NODES_K), jnp.float32),
        mesh=mesh,
        scratch_types=[
            pltpu.VMEM((F_PER_TILE, N_NODES_K), jnp.float32),
            pltpu.VMEM((F_PER_TILE, N_NODES_K), jnp.float32),
            pltpu.VMEM((CH_C,), jnp.int32),
            pltpu.VMEM((CH_C,), jnp.float32),
            pltpu.VMEM((CH_C,), jnp.int32),
            pltpu.VMEM((CH_C,), jnp.float32),
            pltpu.SemaphoreType.DMA,
            pltpu.SemaphoreType.DMA,
        ],
        compiler_params=_SC_PARAMS,
    )
    def k(pk_hbm, xt_hbm, fac_hbm, out_hbm,
          xc, gc, pb0, fb0, pb1, fb1, s0, s1):
        wid = _tile_id()
        pltpu.sync_copy(xt_hbm.at[pl.ds(wid * F_PER_TILE, F_PER_TILE)], xc)
        sgn0 = _sign0(wid)
        f_idx = [jnp.full((LANES,), f, jnp.int32) for f in range(F_PER_TILE)]
        z = jnp.zeros((LANES,), jnp.float32)
        n_ch = N_EDGES_K // CH_C

        def cps(ci, pb, fb, sem):
            base = ci * CH_C
            return (pltpu.make_async_copy(pk_hbm.at[pl.ds(base, CH_C)], pb, sem),
                    pltpu.make_async_copy(fac_hbm.at[pl.ds(base, CH_C)], fb, sem))

        def do_chunk(pb, fb):
            @plsc.parallel_loop(0, CH_C // LANES, unroll=2)
            def _grp(g):
                uu, vv = _unpack_uv(pb[pl.ds(g * LANES, LANES)])
                ff = fb[pl.ds(g * LANES, LANES)]
                fj = ff * sgn0
                cu = plsc.load_gather(xc, [f_idx[0], uu])
                cv = plsc.load_gather(xc, [f_idx[0], vv])
                plsc.addupdate_scatter(gc, [f_idx[0], vv], fj * cu)
                plsc.addupdate_scatter(gc, [f_idx[0], uu], fj * cv)
                for f in range(1, F_PER_TILE):
                    cu = plsc.load_gather(xc, [f_idx[f], uu])
                    cv = plsc.load_gather(xc, [f_idx[f], vv])
                    plsc.addupdate_scatter(gc, [f_idx[f], vv], ff * cu)
                    plsc.addupdate_scatter(gc, [f_idx[f], uu], ff * cv)

        for c in cps(0, pb0, fb0, s0):
            c.start()

        for f in range(F_PER_TILE):
            @pl.loop(0, N_NODES_K // LANES)
            def _zero(i, f=f):
                gc[f, pl.ds(i * LANES, LANES)] = z

        @pl.loop(0, n_ch // 2)
        def _pair(p):
            ci0 = p * 2
            for c in cps(ci0 + 1, pb1, fb1, s1):
                c.start()
            for c in cps(ci0, pb0, fb0, s0):
                c.wait()
            do_chunk(pb0, fb0)

            @pl.when(p + 1 < n_ch // 2)
            def _():
                for c in cps(ci0 + 2, pb0, fb0, s0):
                    c.start()

            for c in cps(ci0 + 1, pb1, fb1, s1):
                c.wait()
            do_chunk(pb1, fb1)

        pltpu.sync_copy(gc, out_hbm.at[pl.ds(wid * F_PER_TILE, F_PER_TILE)])

    return k(pack, xt, factor)


def kernel(x, edges):
    pack = edges[:, 0] * 16384 + edges[:, 1]
    xt = x.T

    innerp_flat = _inner_products(xt, pack)
    factor, energy = _factor_energy(innerp_flat)
    gradt = _scatter_grad(xt, pack, factor)
    return energy[0], gradt.T
